# baseline (speedup 1.0000x reference)
import functools

import jax
import jax.numpy as jnp
from jax import lax
from jax.experimental import pallas as pl
from jax.experimental.pallas import tpu as pltpu
from jax.experimental.pallas import tpu_sc as plsc

N = 10000
E = 320000
NC = 2
NS = 16
CH = 125
NCH32 = E // (NC * NS) // CH
NCH16 = E // NS // CH
SB = 4
SBCH = NCH16 // SB
SB2 = 2
N_PAD = 10240
NPT = N_PAD // NS
BN = 2000

_LEAK = 0.01


def _lrelu(v):
    return jnp.where(v >= 0, v, _LEAK * v)


def _sc_mesh():
    return plsc.VectorSubcoreMesh(core_axis_name="c", subcore_axis_name="s")


def _deg_body(dst_hbm, ones_hbm, z1_hbm, out_hbm, idx_v, ones_v, acc_sh):
    c = lax.axis_index("c")
    s = lax.axis_index("s")
    w = c * NS + s
    pltpu.sync_copy(dst_hbm.at[w], idx_v)
    pltpu.sync_copy(ones_hbm, ones_v)
    pltpu.sync_copy(z1_hbm.at[pl.ds(s * NPT, NPT)], acc_sh.at[pl.ds(s * NPT, NPT)])
    plsc.subcore_barrier()

    def body(j, carry):
        pltpu.sync_copy(ones_v, acc_sh.at[idx_v.at[j]], add=True)
        return carry

    lax.fori_loop(0, NCH32, body, 0)
    plsc.subcore_barrier()
    pltpu.sync_copy(acc_sh.at[pl.ds(s * NPT, NPT)],
                    out_hbm.at[pl.ds(c * N_PAD + s * NPT, NPT)])


def _agg1_body(tbl_hbm, src01_hbm, dst_hbm, z128_hbm, out_hbm,
               isrc_v, idst_v, rows_a, rows_b, acc_sh, sem_a, sem_b,
               sem_sa, sem_sb):
    c = lax.axis_index("c")
    s = lax.axis_index("s")

    pltpu.sync_copy(z128_hbm.at[pl.ds(s * NPT, NPT)], acc_sh.at[pl.ds(s * NPT, NPT)])
    plsc.subcore_barrier()

    def wait_g(buf, sem):
        pltpu.make_async_copy(tbl_hbm.at[isrc_v.at[0]], buf, sem).wait()

    def wait_s(buf, sem):
        pltpu.make_async_copy(buf, acc_sh.at[idst_v.at[0]], sem).wait()

    def body(jj, carry):
        j0 = 2 * jj
        wait_g(rows_a, sem_a)
        pltpu.async_copy(rows_a, acc_sh.at[idst_v.at[j0]], sem_sa, add=True)
        wait_g(rows_b, sem_b)
        pltpu.async_copy(rows_b, acc_sh.at[idst_v.at[j0 + 1]], sem_sb, add=True)
        wait_s(rows_a, sem_sa)
        pltpu.async_copy(tbl_hbm.at[isrc_v.at[j0 + 2]], rows_a, sem_a)
        wait_s(rows_b, sem_sb)
        pltpu.async_copy(tbl_hbm.at[isrc_v.at[j0 + 3]], rows_b, sem_b)
        return carry

    for b in range(SB):
        blk = s * SB + b
        pltpu.sync_copy(src01_hbm.at[c * (NS * SB) + blk], isrc_v)
        pltpu.sync_copy(dst_hbm.at[blk], idst_v)
        pltpu.async_copy(tbl_hbm.at[isrc_v.at[0]], rows_a, sem_a)
        pltpu.async_copy(tbl_hbm.at[isrc_v.at[1]], rows_b, sem_b)
        lax.fori_loop(0, SBCH // 2 - 1, body, 0)
        wait_g(rows_a, sem_a)
        pltpu.async_copy(rows_a, acc_sh.at[idst_v.at[SBCH - 2]], sem_sa, add=True)
        wait_g(rows_b, sem_b)
        pltpu.async_copy(rows_b, acc_sh.at[idst_v.at[SBCH - 1]], sem_sb, add=True)
        wait_s(rows_a, sem_sa)
        wait_s(rows_b, sem_sb)

    plsc.subcore_barrier()
    pltpu.sync_copy(acc_sh.at[pl.ds(s * NPT, NPT)],
                    out_hbm.at[pl.ds(c * N_PAD + s * NPT, NPT)])


def _agg2_body(tbl_hbm, src_hbm, dst_hbm, z128_hbm, out_hbm,
               isrc_v, idst_v, rows_a, rows_b, acc_sh, sem_a, sem_b,
               sem_sa, sem_sb):
    c = lax.axis_index("c")
    s = lax.axis_index("s")
    w = c * NS + s
    pltpu.sync_copy(z128_hbm.at[pl.ds(s * NPT, NPT)], acc_sh.at[pl.ds(s * NPT, NPT)])
    plsc.subcore_barrier()

    def wait_g(buf, sem):
        pltpu.make_async_copy(tbl_hbm.at[isrc_v.at[0]], buf, sem).wait()

    def wait_s(buf, sem):
        pltpu.make_async_copy(buf, acc_sh.at[idst_v.at[0]], sem).wait()

    def body(jj, carry):
        j0 = 2 * jj
        wait_g(rows_a, sem_a)
        pltpu.async_copy(rows_a, acc_sh.at[idst_v.at[j0]], sem_sa, add=True)
        wait_g(rows_b, sem_b)
        pltpu.async_copy(rows_b, acc_sh.at[idst_v.at[j0 + 1]], sem_sb, add=True)
        wait_s(rows_a, sem_sa)
        pltpu.async_copy(tbl_hbm.at[isrc_v.at[j0 + 2]], rows_a, sem_a)
        wait_s(rows_b, sem_sb)
        pltpu.async_copy(tbl_hbm.at[isrc_v.at[j0 + 3]], rows_b, sem_b)
        return carry

    for b in range(SB2):
        blk = w * SB2 + b
        pltpu.sync_copy(src_hbm.at[blk], isrc_v)
        pltpu.sync_copy(dst_hbm.at[blk], idst_v)
        pltpu.async_copy(tbl_hbm.at[isrc_v.at[0]], rows_a, sem_a)
        pltpu.async_copy(tbl_hbm.at[isrc_v.at[1]], rows_b, sem_b)
        lax.fori_loop(0, SBCH // 2 - 1, body, 0)
        wait_g(rows_a, sem_a)
        pltpu.async_copy(rows_a, acc_sh.at[idst_v.at[SBCH - 2]], sem_sa, add=True)
        wait_g(rows_b, sem_b)
        pltpu.async_copy(rows_b, acc_sh.at[idst_v.at[SBCH - 1]], sem_sb, add=True)
        wait_s(rows_a, sem_sa)
        wait_s(rows_b, sem_sb)

    plsc.subcore_barrier()
    pltpu.sync_copy(acc_sh.at[pl.ds(s * NPT, NPT)],
                    out_hbm.at[pl.ds(c * N_PAD + s * NPT, NPT)])


@functools.lru_cache(maxsize=None)
def _build_sc_kernels():
    deg = pl.kernel(
        _deg_body,
        out_type=jax.ShapeDtypeStruct((2 * N_PAD,), jnp.float32),
        mesh=_sc_mesh(),
        scratch_types=[
            pltpu.VMEM((NCH32, CH), jnp.int32),
            pltpu.VMEM((CH,), jnp.float32),
            pltpu.VMEM_SHARED((N_PAD,), jnp.float32),
        ],
    )
    agg1 = pl.kernel(
        _agg1_body,
        out_type=jax.ShapeDtypeStruct((2 * N_PAD, 128), jnp.float32),
        mesh=_sc_mesh(),
        scratch_types=[
            pltpu.VMEM((SBCH, CH), jnp.int32),
            pltpu.VMEM((SBCH, CH), jnp.int32),
            pltpu.VMEM((CH, 128), jnp.float32),
            pltpu.VMEM((CH, 128), jnp.float32),
            pltpu.VMEM_SHARED((N_PAD, 128), jnp.float32),
            pltpu.SemaphoreType.DMA,
            pltpu.SemaphoreType.DMA,
            pltpu.SemaphoreType.DMA,
            pltpu.SemaphoreType.DMA,
        ],
    )
    agg2 = pl.kernel(
        _agg2_body,
        out_type=jax.ShapeDtypeStruct((2 * N_PAD, 128), jnp.float32),
        mesh=_sc_mesh(),
        scratch_types=[
            pltpu.VMEM((SBCH, CH), jnp.int32),
            pltpu.VMEM((SBCH, CH), jnp.int32),
            pltpu.VMEM((CH, 128), jnp.float32),
            pltpu.VMEM((CH, 128), jnp.float32),
            pltpu.VMEM_SHARED((N_PAD, 128), jnp.float32),
            pltpu.SemaphoreType.DMA,
            pltpu.SemaphoreType.DMA,
            pltpu.SemaphoreType.DMA,
            pltpu.SemaphoreType.DMA,
        ],
    )
    return deg, agg1, agg2


def _deg_k(dst32, ones16, z16):
    return _build_sc_kernels()[0](dst32, ones16, z16)


def _agg1_k(tbl, src01, dst16, z128):
    return _build_sc_kernels()[1](tbl, src01, dst16, z128)


def _agg2_k(tbl, src32, dst32, z128):
    return _build_sc_kernels()[2](tbl, src32, dst32, z128)


def _tc1_body(x_ref, w1_ref, b1_ref, w2_ref, b2_ref, wc1_ref, degr_ref,
              tbl_ref, dinv_ref):
    h = jnp.dot(x_ref[...], w1_ref[...], preferred_element_type=jnp.float32)
    h = _lrelu(h + b1_ref[...])
    h = jnp.dot(h, w2_ref[...], preferred_element_type=jnp.float32)
    h = _lrelu(h + b2_ref[...])
    hw = jnp.dot(h, wc1_ref[...], preferred_element_type=jnp.float32)
    deg = degr_ref[0] + degr_ref[1] + 1.0
    dinv = lax.rsqrt(deg)
    hws = hw * dinv
    dinv_ref[...] = dinv
    tbl_ref[0, :, :] = hws[:, :128]
    tbl_ref[1, :, :] = hws[:, 128:]


def _tc1(x, w1, b1, w2, b2, wc1, degr3):
    grid = (N // BN,)
    return pl.pallas_call(
        _tc1_body,
        grid=grid,
        in_specs=[
            pl.BlockSpec((BN, 128), lambda i: (i, 0)),
            pl.BlockSpec((128, 256), lambda i: (0, 0)),
            pl.BlockSpec((1, 256), lambda i: (0, 0)),
            pl.BlockSpec((256, 256), lambda i: (0, 0)),
            pl.BlockSpec((1, 256), lambda i: (0, 0)),
            pl.BlockSpec((256, 256), lambda i: (0, 0)),
            pl.BlockSpec((2, BN, 1), lambda i: (0, i, 0)),
        ],
        out_specs=[
            pl.BlockSpec((2, BN, 128), lambda i: (0, i, 0)),
            pl.BlockSpec((BN, 1), lambda i: (i, 0)),
        ],
        out_shape=[
            jax.ShapeDtypeStruct((2, N, 128), jnp.float32),
            jax.ShapeDtypeStruct((N, 1), jnp.float32),
        ],
    )(x, w1, b1, w2, b2, wc1, degr3)


def _tc2_body(agg_ref, tbl_ref, dinv_ref, bc1_ref, wc2_ref, emb1_ref, tbl2_ref):
    a = jnp.concatenate(
        [agg_ref[0] + tbl_ref[0], agg_ref[1] + tbl_ref[1]], axis=1)
    dinv = dinv_ref[...]
    h = _lrelu(a * dinv + bc1_ref[...])
    nrm = jnp.sqrt(jnp.sum(h * h, axis=1, keepdims=True))
    emb1 = h / nrm
    emb1_ref[...] = emb1
    hw2 = jnp.dot(emb1, wc2_ref[...], preferred_element_type=jnp.float32)
    tbl2_ref[...] = hw2 * dinv


def _tc2(agg3, tbl3, dinv, bc1, wc2):
    grid = (N // BN,)
    return pl.pallas_call(
        _tc2_body,
        grid=grid,
        in_specs=[
            pl.BlockSpec((2, BN, 128), lambda i: (0, i, 0)),
            pl.BlockSpec((2, BN, 128), lambda i: (0, i, 0)),
            pl.BlockSpec((BN, 1), lambda i: (i, 0)),
            pl.BlockSpec((1, 256), lambda i: (0, 0)),
            pl.BlockSpec((256, 128), lambda i: (0, 0)),
        ],
        out_specs=[
            pl.BlockSpec((BN, 256), lambda i: (i, 0)),
            pl.BlockSpec((BN, 128), lambda i: (i, 0)),
        ],
        out_shape=[
            jax.ShapeDtypeStruct((N, 256), jnp.float32),
            jax.ShapeDtypeStruct((N, 128), jnp.float32),
        ],
    )(agg3, tbl3, dinv, bc1, wc2)


def _tc3_body(agg_ref, tbl2_ref, dinv_ref, bc2_ref, emb2_ref):
    full = (agg_ref[0] + agg_ref[1] + tbl2_ref[...]) * dinv_ref[...] + bc2_ref[...]
    h = _lrelu(full)
    nrm = jnp.sqrt(jnp.sum(h * h, axis=1, keepdims=True))
    emb2_ref[...] = h / nrm


def _tc3(agg3, tbl2, dinv, bc2):
    grid = (N // BN,)
    return pl.pallas_call(
        _tc3_body,
        grid=grid,
        in_specs=[
            pl.BlockSpec((2, BN, 128), lambda i: (0, i, 0)),
            pl.BlockSpec((BN, 128), lambda i: (i, 0)),
            pl.BlockSpec((BN, 1), lambda i: (i, 0)),
            pl.BlockSpec((1, 128), lambda i: (0, 0)),
        ],
        out_specs=pl.BlockSpec((BN, 128), lambda i: (i, 0)),
        out_shape=jax.ShapeDtypeStruct((N, 128), jnp.float32),
    )(agg3, tbl2, dinv, bc2)


def kernel(x, edge_index, W_pre1, b_pre1, W_pre2, b_pre2, W_c1, b_c1, W_c2, b_c2):
    src = edge_index[0]
    dst = edge_index[1]
    dst32 = dst.reshape(NC * NS, NCH32, CH)
    src32b = src.reshape(NC * NS * SB2, SBCH, CH)
    dst32b = dst.reshape(NC * NS * SB2, SBCH, CH)
    src01 = jnp.stack([src, src + N]).reshape(2 * NS * SB, SBCH, CH)
    dst16 = dst.reshape(NS * SB, SBCH, CH)
    ones1 = jnp.ones((CH,), jnp.float32)
    z1 = jnp.zeros((N_PAD,), jnp.float32)
    z128 = jnp.zeros((N_PAD, 128), jnp.float32)
    b1 = b_pre1.reshape(1, -1)
    b2 = b_pre2.reshape(1, -1)
    bc1 = b_c1.reshape(1, -1)
    bc2 = b_c2.reshape(1, -1)

    degr = _deg_k(dst32, ones1, z1)
    tbl1_3, dinv = _tc1(x, W_pre1, b1, W_pre2, b2, W_c1,
                        degr.reshape(2, N_PAD, 1)[:, :N])
    agg1 = _agg1_k(tbl1_3.reshape(2 * N, 128),
                   src01, dst16, z128)
    emb1, tbl2 = _tc2(agg1.reshape(2, N_PAD, 128)[:, :N], tbl1_3, dinv, bc1, W_c2)
    agg2 = _agg2_k(tbl2, src32b, dst32b, z128)
    emb2 = _tc3(agg2.reshape(2, N_PAD, 128)[:, :N], tbl2, dinv, bc2)
    return emb2, emb1, emb2


# speedup vs baseline: 1.2401x; 1.2401x over previous
import functools

import jax
import jax.numpy as jnp
from jax import lax
from jax.experimental import pallas as pl
from jax.experimental.pallas import tpu as pltpu
from jax.experimental.pallas import tpu_sc as plsc

N = 10000
E = 320000
NC = 2
NS = 16
CH = 125
NCH32 = E // (NC * NS) // CH
NCH16 = E // NS // CH
SB = 4
SBCH = NCH16 // SB
SB2 = 2
N_PAD = 10240
NPT = N_PAD // NS
BN = 2000

_LEAK = 0.01


def _lrelu(v):
    return jnp.where(v >= 0, v, _LEAK * v)


def _sc_mesh():
    return plsc.VectorSubcoreMesh(core_axis_name="c", subcore_axis_name="s")


def _deg_body(dst_hbm, ones_hbm, z1_hbm, out_hbm, idx_v, ones_v, acc_sh):
    c = lax.axis_index("c")
    s = lax.axis_index("s")
    w = c * NS + s
    pltpu.sync_copy(dst_hbm.at[w], idx_v)
    pltpu.sync_copy(ones_hbm, ones_v)
    pltpu.sync_copy(z1_hbm.at[pl.ds(s * NPT, NPT)], acc_sh.at[pl.ds(s * NPT, NPT)])
    plsc.subcore_barrier()

    def body(j, carry):
        pltpu.sync_copy(ones_v, acc_sh.at[idx_v.at[j]], add=True)
        return carry

    lax.fori_loop(0, NCH32, body, 0)
    plsc.subcore_barrier()
    pltpu.sync_copy(acc_sh.at[pl.ds(s * NPT, NPT)],
                    out_hbm.at[pl.ds(c * N_PAD + s * NPT, NPT)])


def _agg1_body(tbl_hbm, src01_hbm, dst_hbm, z128_hbm, out_hbm,
               isrc_v, idst_v, rows_a, rows_b, acc_sh, sem_a, sem_b):
    c = lax.axis_index("c")
    s = lax.axis_index("s")

    pltpu.sync_copy(z128_hbm.at[pl.ds(s * NPT, NPT)], acc_sh.at[pl.ds(s * NPT, NPT)])
    plsc.subcore_barrier()

    def wait_a():
        pltpu.make_async_copy(tbl_hbm.at[isrc_v.at[0]], rows_a, sem_a).wait()

    def wait_b():
        pltpu.make_async_copy(tbl_hbm.at[isrc_v.at[0]], rows_b, sem_b).wait()

    def body(jj, carry):
        j0 = 2 * jj
        wait_a()
        pltpu.sync_copy(rows_a, acc_sh.at[idst_v.at[j0]], add=True)
        pltpu.async_copy(tbl_hbm.at[isrc_v.at[j0 + 2]], rows_a, sem_a)
        wait_b()
        pltpu.sync_copy(rows_b, acc_sh.at[idst_v.at[j0 + 1]], add=True)
        pltpu.async_copy(tbl_hbm.at[isrc_v.at[j0 + 3]], rows_b, sem_b)
        return carry

    for b in range(SB):
        blk = s * SB + b
        pltpu.sync_copy(src01_hbm.at[c * (NS * SB) + blk], isrc_v)
        pltpu.sync_copy(dst_hbm.at[blk], idst_v)
        pltpu.async_copy(tbl_hbm.at[isrc_v.at[0]], rows_a, sem_a)
        pltpu.async_copy(tbl_hbm.at[isrc_v.at[1]], rows_b, sem_b)
        lax.fori_loop(0, SBCH // 2 - 1, body, 0)
        wait_a()
        pltpu.sync_copy(rows_a, acc_sh.at[idst_v.at[SBCH - 2]], add=True)
        wait_b()
        pltpu.sync_copy(rows_b, acc_sh.at[idst_v.at[SBCH - 1]], add=True)

    plsc.subcore_barrier()
    pltpu.sync_copy(acc_sh.at[pl.ds(s * NPT, NPT)],
                    out_hbm.at[pl.ds(c * N_PAD + s * NPT, NPT)])


def _agg2_body(tbl_hbm, src_hbm, dst_hbm, z128_hbm, out_hbm,
               isrc_v, idst_v, rows_a, rows_b, acc_sh, sem_a, sem_b):
    c = lax.axis_index("c")
    s = lax.axis_index("s")
    w = c * NS + s
    pltpu.sync_copy(z128_hbm.at[pl.ds(s * NPT, NPT)], acc_sh.at[pl.ds(s * NPT, NPT)])
    plsc.subcore_barrier()

    def wait_a():
        pltpu.make_async_copy(tbl_hbm.at[isrc_v.at[0]], rows_a, sem_a).wait()

    def wait_b():
        pltpu.make_async_copy(tbl_hbm.at[isrc_v.at[0]], rows_b, sem_b).wait()

    def body(jj, carry):
        j0 = 2 * jj
        wait_a()
        pltpu.sync_copy(rows_a, acc_sh.at[idst_v.at[j0]], add=True)
        pltpu.async_copy(tbl_hbm.at[isrc_v.at[j0 + 2]], rows_a, sem_a)
        wait_b()
        pltpu.sync_copy(rows_b, acc_sh.at[idst_v.at[j0 + 1]], add=True)
        pltpu.async_copy(tbl_hbm.at[isrc_v.at[j0 + 3]], rows_b, sem_b)
        return carry

    for b in range(SB2):
        blk = w * SB2 + b
        pltpu.sync_copy(src_hbm.at[blk], isrc_v)
        pltpu.sync_copy(dst_hbm.at[blk], idst_v)
        pltpu.async_copy(tbl_hbm.at[isrc_v.at[0]], rows_a, sem_a)
        pltpu.async_copy(tbl_hbm.at[isrc_v.at[1]], rows_b, sem_b)
        lax.fori_loop(0, SBCH // 2 - 1, body, 0)
        wait_a()
        pltpu.sync_copy(rows_a, acc_sh.at[idst_v.at[SBCH - 2]], add=True)
        wait_b()
        pltpu.sync_copy(rows_b, acc_sh.at[idst_v.at[SBCH - 1]], add=True)

    plsc.subcore_barrier()
    pltpu.sync_copy(acc_sh.at[pl.ds(s * NPT, NPT)],
                    out_hbm.at[pl.ds(c * N_PAD + s * NPT, NPT)])


@functools.lru_cache(maxsize=None)
def _build_sc_kernels():
    deg = pl.kernel(
        _deg_body,
        out_type=jax.ShapeDtypeStruct((2 * N_PAD,), jnp.float32),
        mesh=_sc_mesh(),
        scratch_types=[
            pltpu.VMEM((NCH32, CH), jnp.int32),
            pltpu.VMEM((CH,), jnp.float32),
            pltpu.VMEM_SHARED((N_PAD,), jnp.float32),
        ],
    )
    agg1 = pl.kernel(
        _agg1_body,
        out_type=jax.ShapeDtypeStruct((2 * N_PAD, 128), jnp.float32),
        mesh=_sc_mesh(),
        scratch_types=[
            pltpu.VMEM((SBCH, CH), jnp.int32),
            pltpu.VMEM((SBCH, CH), jnp.int32),
            pltpu.VMEM((CH, 128), jnp.float32),
            pltpu.VMEM((CH, 128), jnp.float32),
            pltpu.VMEM_SHARED((N_PAD, 128), jnp.float32),
            pltpu.SemaphoreType.DMA,
            pltpu.SemaphoreType.DMA,
        ],
    )
    agg2 = pl.kernel(
        _agg2_body,
        out_type=jax.ShapeDtypeStruct((2 * N_PAD, 128), jnp.float32),
        mesh=_sc_mesh(),
        scratch_types=[
            pltpu.VMEM((SBCH, CH), jnp.int32),
            pltpu.VMEM((SBCH, CH), jnp.int32),
            pltpu.VMEM((CH, 128), jnp.float32),
            pltpu.VMEM((CH, 128), jnp.float32),
            pltpu.VMEM_SHARED((N_PAD, 128), jnp.float32),
            pltpu.SemaphoreType.DMA,
            pltpu.SemaphoreType.DMA,
        ],
    )
    return deg, agg1, agg2


def _deg_k(dst32, ones16, z16):
    return _build_sc_kernels()[0](dst32, ones16, z16)


def _agg1_k(tbl, src01, dst16, z128):
    return _build_sc_kernels()[1](tbl, src01, dst16, z128)


def _agg2_k(tbl, src32, dst32, z128):
    return _build_sc_kernels()[2](tbl, src32, dst32, z128)


def _tc1_body(x_ref, w1_ref, b1_ref, w2_ref, b2_ref, wc1_ref, degr_ref,
              tbl_ref, dinv_ref):
    h = jnp.dot(x_ref[...], w1_ref[...], preferred_element_type=jnp.float32)
    h = _lrelu(h + b1_ref[...])
    h = jnp.dot(h, w2_ref[...], preferred_element_type=jnp.float32)
    h = _lrelu(h + b2_ref[...])
    hw = jnp.dot(h, wc1_ref[...], preferred_element_type=jnp.float32)
    deg = degr_ref[0] + degr_ref[1] + 1.0
    dinv = lax.rsqrt(deg)
    hws = hw * dinv
    dinv_ref[...] = dinv
    tbl_ref[0, :, :] = hws[:, :128]
    tbl_ref[1, :, :] = hws[:, 128:]


def _tc1(x, w1, b1, w2, b2, wc1, degr3):
    grid = (N // BN,)
    return pl.pallas_call(
        _tc1_body,
        grid=grid,
        in_specs=[
            pl.BlockSpec((BN, 128), lambda i: (i, 0)),
            pl.BlockSpec((128, 256), lambda i: (0, 0)),
            pl.BlockSpec((1, 256), lambda i: (0, 0)),
            pl.BlockSpec((256, 256), lambda i: (0, 0)),
            pl.BlockSpec((1, 256), lambda i: (0, 0)),
            pl.BlockSpec((256, 256), lambda i: (0, 0)),
            pl.BlockSpec((2, BN, 1), lambda i: (0, i, 0)),
        ],
        out_specs=[
            pl.BlockSpec((2, BN, 128), lambda i: (0, i, 0)),
            pl.BlockSpec((BN, 1), lambda i: (i, 0)),
        ],
        out_shape=[
            jax.ShapeDtypeStruct((2, N, 128), jnp.float32),
            jax.ShapeDtypeStruct((N, 1), jnp.float32),
        ],
    )(x, w1, b1, w2, b2, wc1, degr3)


def _tc2_body(agg_ref, tbl_ref, dinv_ref, bc1_ref, wc2_ref, emb1_ref, tbl2_ref):
    a = jnp.concatenate(
        [agg_ref[0] + tbl_ref[0], agg_ref[1] + tbl_ref[1]], axis=1)
    dinv = dinv_ref[...]
    h = _lrelu(a * dinv + bc1_ref[...])
    nrm = jnp.sqrt(jnp.sum(h * h, axis=1, keepdims=True))
    emb1 = h / nrm
    emb1_ref[...] = emb1
    hw2 = jnp.dot(emb1, wc2_ref[...], preferred_element_type=jnp.float32)
    tbl2_ref[...] = hw2 * dinv


def _tc2(agg3, tbl3, dinv, bc1, wc2):
    grid = (N // BN,)
    return pl.pallas_call(
        _tc2_body,
        grid=grid,
        in_specs=[
            pl.BlockSpec((2, BN, 128), lambda i: (0, i, 0)),
            pl.BlockSpec((2, BN, 128), lambda i: (0, i, 0)),
            pl.BlockSpec((BN, 1), lambda i: (i, 0)),
            pl.BlockSpec((1, 256), lambda i: (0, 0)),
            pl.BlockSpec((256, 128), lambda i: (0, 0)),
        ],
        out_specs=[
            pl.BlockSpec((BN, 256), lambda i: (i, 0)),
            pl.BlockSpec((BN, 128), lambda i: (i, 0)),
        ],
        out_shape=[
            jax.ShapeDtypeStruct((N, 256), jnp.float32),
            jax.ShapeDtypeStruct((N, 128), jnp.float32),
        ],
    )(agg3, tbl3, dinv, bc1, wc2)


def _tc3_body(agg_ref, tbl2_ref, dinv_ref, bc2_ref, emb2_ref):
    full = (agg_ref[0] + agg_ref[1] + tbl2_ref[...]) * dinv_ref[...] + bc2_ref[...]
    h = _lrelu(full)
    nrm = jnp.sqrt(jnp.sum(h * h, axis=1, keepdims=True))
    emb2_ref[...] = h / nrm


def _tc3(agg3, tbl2, dinv, bc2):
    grid = (N // BN,)
    return pl.pallas_call(
        _tc3_body,
        grid=grid,
        in_specs=[
            pl.BlockSpec((2, BN, 128), lambda i: (0, i, 0)),
            pl.BlockSpec((BN, 128), lambda i: (i, 0)),
            pl.BlockSpec((BN, 1), lambda i: (i, 0)),
            pl.BlockSpec((1, 128), lambda i: (0, 0)),
        ],
        out_specs=pl.BlockSpec((BN, 128), lambda i: (i, 0)),
        out_shape=jax.ShapeDtypeStruct((N, 128), jnp.float32),
    )(agg3, tbl2, dinv, bc2)


def kernel(x, edge_index, W_pre1, b_pre1, W_pre2, b_pre2, W_c1, b_c1, W_c2, b_c2):
    src = edge_index[0]
    dst = edge_index[1]
    dst32 = dst.reshape(NC * NS, NCH32, CH)
    src32b = src.reshape(NC * NS * SB2, SBCH, CH)
    dst32b = dst.reshape(NC * NS * SB2, SBCH, CH)
    src01 = jnp.stack([src, src + N]).reshape(2 * NS * SB, SBCH, CH)
    dst16 = dst.reshape(NS * SB, SBCH, CH)
    ones1 = jnp.ones((CH,), jnp.float32)
    z1 = jnp.zeros((N_PAD,), jnp.float32)
    z128 = jnp.zeros((N_PAD, 128), jnp.float32)
    b1 = b_pre1.reshape(1, -1)
    b2 = b_pre2.reshape(1, -1)
    bc1 = b_c1.reshape(1, -1)
    bc2 = b_c2.reshape(1, -1)

    degr = _deg_k(dst32, ones1, z1)
    tbl1_3, dinv = _tc1(x, W_pre1, b1, W_pre2, b2, W_c1,
                        degr.reshape(2, N_PAD, 1)[:, :N])
    agg1 = _agg1_k(tbl1_3.reshape(2 * N, 128),
                   src01, dst16, z128)
    emb1, tbl2 = _tc2(agg1.reshape(2, N_PAD, 128)[:, :N], tbl1_3, dinv, bc1, W_c2)
    agg2 = _agg2_k(tbl2, src32b, dst32b, z128)
    emb2 = _tc3(agg2.reshape(2, N_PAD, 128)[:, :N], tbl2, dinv, bc2)
    return emb2, emb1, emb2


# speedup vs baseline: 1.2818x; 1.0337x over previous
import functools

import jax
import jax.numpy as jnp
from jax import lax
from jax.experimental import pallas as pl
from jax.experimental.pallas import tpu as pltpu
from jax.experimental.pallas import tpu_sc as plsc

N = 10000
E = 320000
NC = 2
NS = 16
CH = 125
NCH32 = E // (NC * NS) // CH
NCH16 = E // NS // CH
SB = 4
SBCH = NCH16 // SB
SB2 = 2
N_PAD = 10240
NPT = 640
WST = 624
BN = 2000

_LEAK = 0.01


def _lrelu(v):
    return jnp.where(v >= 0, v, _LEAK * v)


def _sc_mesh():
    return plsc.VectorSubcoreMesh(core_axis_name="c", subcore_axis_name="s")


def _deg_body(dst_hbm, ones_hbm, z1_hbm, out_hbm, idx_v, ones_v, acc_sh):
    c = lax.axis_index("c")
    s = lax.axis_index("s")
    w = c * NS + s
    pltpu.sync_copy(dst_hbm.at[w], idx_v)
    pltpu.sync_copy(ones_hbm, ones_v)
    pltpu.sync_copy(z1_hbm.at[pl.ds(s * NPT, NPT)], acc_sh.at[pl.ds(s * NPT, NPT)])
    plsc.subcore_barrier()

    def body(j, carry):
        pltpu.sync_copy(ones_v, acc_sh.at[idx_v.at[j]], add=True)
        return carry

    lax.fori_loop(0, NCH32, body, 0)
    plsc.subcore_barrier()
    pltpu.sync_copy(acc_sh.at[pl.ds(s * NPT, NPT)],
                    out_hbm.at[pl.ds(c * N_PAD + s * NPT, NPT)])


def _agg1_body(tbl_hbm, src01_hbm, dst_hbm, z128_hbm, out_hbm,
               isrc_v, idst_v, rows_a, rows_b, acc_sh, sem_a, sem_b):
    c = lax.axis_index("c")
    s = lax.axis_index("s")

    pltpu.sync_copy(z128_hbm.at[pl.ds(s * WST, NPT)], acc_sh.at[pl.ds(s * WST, NPT)])
    plsc.subcore_barrier()

    def wait_a():
        pltpu.make_async_copy(tbl_hbm.at[isrc_v.at[0]], rows_a, sem_a).wait()

    def wait_b():
        pltpu.make_async_copy(tbl_hbm.at[isrc_v.at[0]], rows_b, sem_b).wait()

    def body(jj, carry):
        j0 = 2 * jj
        wait_a()
        pltpu.sync_copy(rows_a, acc_sh.at[idst_v.at[j0]], add=True)
        pltpu.async_copy(tbl_hbm.at[isrc_v.at[j0 + 2]], rows_a, sem_a)
        wait_b()
        pltpu.sync_copy(rows_b, acc_sh.at[idst_v.at[j0 + 1]], add=True)
        pltpu.async_copy(tbl_hbm.at[isrc_v.at[j0 + 3]], rows_b, sem_b)
        return carry

    for b in range(SB):
        blk = s * SB + b
        pltpu.sync_copy(src01_hbm.at[c * (NS * SB) + blk], isrc_v)
        pltpu.sync_copy(dst_hbm.at[blk], idst_v)
        pltpu.async_copy(tbl_hbm.at[isrc_v.at[0]], rows_a, sem_a)
        pltpu.async_copy(tbl_hbm.at[isrc_v.at[1]], rows_b, sem_b)
        lax.fori_loop(0, SBCH // 2 - 1, body, 0)
        wait_a()
        pltpu.sync_copy(rows_a, acc_sh.at[idst_v.at[SBCH - 2]], add=True)
        wait_b()
        pltpu.sync_copy(rows_b, acc_sh.at[idst_v.at[SBCH - 1]], add=True)

    plsc.subcore_barrier()
    pltpu.sync_copy(acc_sh.at[pl.ds(s * WST, NPT)],
                    out_hbm.at[pl.ds(c * N + s * WST, NPT)])


def _agg2_body(tbl_hbm, src_hbm, dst_hbm, z128_hbm, out_hbm,
               isrc_v, idst_v, rows_a, rows_b, acc_sh, sem_a, sem_b):
    c = lax.axis_index("c")
    s = lax.axis_index("s")
    w = c * NS + s
    pltpu.sync_copy(z128_hbm.at[pl.ds(s * WST, NPT)], acc_sh.at[pl.ds(s * WST, NPT)])
    plsc.subcore_barrier()

    def wait_a():
        pltpu.make_async_copy(tbl_hbm.at[isrc_v.at[0]], rows_a, sem_a).wait()

    def wait_b():
        pltpu.make_async_copy(tbl_hbm.at[isrc_v.at[0]], rows_b, sem_b).wait()

    def body(jj, carry):
        j0 = 2 * jj
        wait_a()
        pltpu.sync_copy(rows_a, acc_sh.at[idst_v.at[j0]], add=True)
        pltpu.async_copy(tbl_hbm.at[isrc_v.at[j0 + 2]], rows_a, sem_a)
        wait_b()
        pltpu.sync_copy(rows_b, acc_sh.at[idst_v.at[j0 + 1]], add=True)
        pltpu.async_copy(tbl_hbm.at[isrc_v.at[j0 + 3]], rows_b, sem_b)
        return carry

    for b in range(SB2):
        blk = w * SB2 + b
        pltpu.sync_copy(src_hbm.at[blk], isrc_v)
        pltpu.sync_copy(dst_hbm.at[blk], idst_v)
        pltpu.async_copy(tbl_hbm.at[isrc_v.at[0]], rows_a, sem_a)
        pltpu.async_copy(tbl_hbm.at[isrc_v.at[1]], rows_b, sem_b)
        lax.fori_loop(0, SBCH // 2 - 1, body, 0)
        wait_a()
        pltpu.sync_copy(rows_a, acc_sh.at[idst_v.at[SBCH - 2]], add=True)
        wait_b()
        pltpu.sync_copy(rows_b, acc_sh.at[idst_v.at[SBCH - 1]], add=True)

    plsc.subcore_barrier()
    pltpu.sync_copy(acc_sh.at[pl.ds(s * WST, NPT)],
                    out_hbm.at[pl.ds(c * N + s * WST, NPT)])


@functools.lru_cache(maxsize=None)
def _build_sc_kernels():
    deg = pl.kernel(
        _deg_body,
        out_type=jax.ShapeDtypeStruct((2 * N_PAD,), jnp.float32),
        mesh=_sc_mesh(),
        scratch_types=[
            pltpu.VMEM((NCH32, CH), jnp.int32),
            pltpu.VMEM((CH,), jnp.float32),
            pltpu.VMEM_SHARED((N_PAD,), jnp.float32),
        ],
    )
    agg1 = pl.kernel(
        _agg1_body,
        out_type=jax.ShapeDtypeStruct((2 * N, 128), jnp.float32),
        mesh=_sc_mesh(),
        scratch_types=[
            pltpu.VMEM((SBCH, CH), jnp.int32),
            pltpu.VMEM((SBCH, CH), jnp.int32),
            pltpu.VMEM((CH, 128), jnp.float32),
            pltpu.VMEM((CH, 128), jnp.float32),
            pltpu.VMEM_SHARED((N, 128), jnp.float32),
            pltpu.SemaphoreType.DMA,
            pltpu.SemaphoreType.DMA,
        ],
    )
    agg2 = pl.kernel(
        _agg2_body,
        out_type=jax.ShapeDtypeStruct((2 * N, 128), jnp.float32),
        mesh=_sc_mesh(),
        scratch_types=[
            pltpu.VMEM((SBCH, CH), jnp.int32),
            pltpu.VMEM((SBCH, CH), jnp.int32),
            pltpu.VMEM((CH, 128), jnp.float32),
            pltpu.VMEM((CH, 128), jnp.float32),
            pltpu.VMEM_SHARED((N, 128), jnp.float32),
            pltpu.SemaphoreType.DMA,
            pltpu.SemaphoreType.DMA,
        ],
    )
    return deg, agg1, agg2


def _deg_k(dst32, ones16, z16):
    return _build_sc_kernels()[0](dst32, ones16, z16)


def _agg1_k(tbl, src01, dst16, z128):
    return _build_sc_kernels()[1](tbl, src01, dst16, z128)


def _agg2_k(tbl, src32, dst32, z128):
    return _build_sc_kernels()[2](tbl, src32, dst32, z128)


def _tc1_body(x_ref, w1_ref, b1_ref, w2_ref, b2_ref, wc1_ref, degr_ref,
              tbl_ref, dinv_ref):
    h = jnp.dot(x_ref[...], w1_ref[...], preferred_element_type=jnp.float32)
    h = _lrelu(h + b1_ref[...])
    h = jnp.dot(h, w2_ref[...], preferred_element_type=jnp.float32)
    h = _lrelu(h + b2_ref[...])
    hw = jnp.dot(h, wc1_ref[...], preferred_element_type=jnp.float32)
    deg = degr_ref[0] + degr_ref[1] + 1.0
    dinv = lax.rsqrt(deg)
    hws = hw * dinv
    dinv_ref[...] = dinv
    tbl_ref[0, :, :] = hws[:, :128]
    tbl_ref[1, :, :] = hws[:, 128:]


def _tc1(x, w1, b1, w2, b2, wc1, degr3):
    grid = (N // BN,)
    return pl.pallas_call(
        _tc1_body,
        grid=grid,
        in_specs=[
            pl.BlockSpec((BN, 128), lambda i: (i, 0)),
            pl.BlockSpec((128, 256), lambda i: (0, 0)),
            pl.BlockSpec((1, 256), lambda i: (0, 0)),
            pl.BlockSpec((256, 256), lambda i: (0, 0)),
            pl.BlockSpec((1, 256), lambda i: (0, 0)),
            pl.BlockSpec((256, 256), lambda i: (0, 0)),
            pl.BlockSpec((2, BN, 1), lambda i: (0, i, 0)),
        ],
        out_specs=[
            pl.BlockSpec((2, BN, 128), lambda i: (0, i, 0)),
            pl.BlockSpec((BN, 1), lambda i: (i, 0)),
        ],
        out_shape=[
            jax.ShapeDtypeStruct((2, N, 128), jnp.float32),
            jax.ShapeDtypeStruct((N, 1), jnp.float32),
        ],
    )(x, w1, b1, w2, b2, wc1, degr3)


def _tc2_body(agg_ref, tbl_ref, dinv_ref, bc1_ref, wc2_ref, emb1_ref, tbl2_ref):
    a = jnp.concatenate(
        [agg_ref[0] + tbl_ref[0], agg_ref[1] + tbl_ref[1]], axis=1)
    dinv = dinv_ref[...]
    h = _lrelu(a * dinv + bc1_ref[...])
    nrm = jnp.sqrt(jnp.sum(h * h, axis=1, keepdims=True))
    emb1 = h / nrm
    emb1_ref[...] = emb1
    hw2 = jnp.dot(emb1, wc2_ref[...], preferred_element_type=jnp.float32)
    tbl2_ref[...] = hw2 * dinv


def _tc2(agg3, tbl3, dinv, bc1, wc2):
    grid = (N // BN,)
    return pl.pallas_call(
        _tc2_body,
        grid=grid,
        in_specs=[
            pl.BlockSpec((2, BN, 128), lambda i: (0, i, 0)),
            pl.BlockSpec((2, BN, 128), lambda i: (0, i, 0)),
            pl.BlockSpec((BN, 1), lambda i: (i, 0)),
            pl.BlockSpec((1, 256), lambda i: (0, 0)),
            pl.BlockSpec((256, 128), lambda i: (0, 0)),
        ],
        out_specs=[
            pl.BlockSpec((BN, 256), lambda i: (i, 0)),
            pl.BlockSpec((BN, 128), lambda i: (i, 0)),
        ],
        out_shape=[
            jax.ShapeDtypeStruct((N, 256), jnp.float32),
            jax.ShapeDtypeStruct((N, 128), jnp.float32),
        ],
    )(agg3, tbl3, dinv, bc1, wc2)


def _tc3_body(agg_ref, tbl2_ref, dinv_ref, bc2_ref, emb2_ref):
    full = (agg_ref[0] + agg_ref[1] + tbl2_ref[...]) * dinv_ref[...] + bc2_ref[...]
    h = _lrelu(full)
    nrm = jnp.sqrt(jnp.sum(h * h, axis=1, keepdims=True))
    emb2_ref[...] = h / nrm


def _tc3(agg3, tbl2, dinv, bc2):
    grid = (N // BN,)
    return pl.pallas_call(
        _tc3_body,
        grid=grid,
        in_specs=[
            pl.BlockSpec((2, BN, 128), lambda i: (0, i, 0)),
            pl.BlockSpec((BN, 128), lambda i: (i, 0)),
            pl.BlockSpec((BN, 1), lambda i: (i, 0)),
            pl.BlockSpec((1, 128), lambda i: (0, 0)),
        ],
        out_specs=pl.BlockSpec((BN, 128), lambda i: (i, 0)),
        out_shape=jax.ShapeDtypeStruct((N, 128), jnp.float32),
    )(agg3, tbl2, dinv, bc2)


def kernel(x, edge_index, W_pre1, b_pre1, W_pre2, b_pre2, W_c1, b_c1, W_c2, b_c2):
    src = edge_index[0]
    dst = edge_index[1]
    dst32 = dst.reshape(NC * NS, NCH32, CH)
    src32b = src.reshape(NC * NS * SB2, SBCH, CH)
    dst32b = dst.reshape(NC * NS * SB2, SBCH, CH)
    src01 = jnp.stack([src, src + N]).reshape(2 * NS * SB, SBCH, CH)
    dst16 = dst.reshape(NS * SB, SBCH, CH)
    ones1 = jnp.ones((CH,), jnp.float32)
    z1 = jnp.zeros((N_PAD,), jnp.float32)
    z128 = jnp.zeros((N, 128), jnp.float32)
    b1 = b_pre1.reshape(1, -1)
    b2 = b_pre2.reshape(1, -1)
    bc1 = b_c1.reshape(1, -1)
    bc2 = b_c2.reshape(1, -1)

    degr = _deg_k(dst32, ones1, z1)
    tbl1_3, dinv = _tc1(x, W_pre1, b1, W_pre2, b2, W_c1,
                        degr.reshape(2, N_PAD, 1)[:, :N])
    agg1 = _agg1_k(tbl1_3.reshape(2 * N, 128),
                   src01, dst16, z128)
    emb1, tbl2 = _tc2(agg1.reshape(2, N, 128), tbl1_3, dinv, bc1, W_c2)
    agg2 = _agg2_k(tbl2, src32b, dst32b, z128)
    emb2 = _tc3(agg2.reshape(2, N, 128), tbl2, dinv, bc2)
    return emb2, emb1, emb2
